# baseline (speedup 1.0000x reference)
import functools

import jax
import jax.numpy as jnp
from jax import lax
from jax.experimental import pallas as pl
from jax.experimental.pallas import tpu as pltpu
from jax.experimental.pallas import tpu_sc as plsc

V = 10000
D_EMB = 256
D_HID = 512
B = 8
CTX = 4
SRC = 256
QRY = 128
TGT = 64
L_CAT = SRC + QRY

NC = 2
NS = 16
NW = NC * NS
LN = 16

VPAD = 10240
BLKV = 1024
NBLK = VPAD // BLKV

_PREC = jax.lax.Precision.DEFAULT

def _wid():
    return lax.axis_index("s") * NC + lax.axis_index("c")


@functools.cache
def _make_sc_rep_sums():
    mesh = plsc.VectorSubcoreMesh(core_axis_name="c", subcore_axis_name="s")
    return functools.partial(
        pl.kernel,
        out_type=(
            jax.ShapeDtypeStruct((40 * D_EMB,), jnp.float32),
            jax.ShapeDtypeStruct((40 * LN,), jnp.float32),
        ),
        mesh=mesh,
        scratch_types=[
            pltpu.VMEM((SRC,), jnp.int32),
            pltpu.VMEM((SRC,), jnp.int32),
            pltpu.VMEM((128, D_EMB), jnp.float32),
            pltpu.VMEM((128, D_EMB), jnp.float32),
            pltpu.VMEM((D_EMB,), jnp.float32),
            pltpu.VMEM((LN,), jnp.float32),
            pltpu.SemaphoreType.DMA,
        ],
        compiler_params=pltpu.CompilerParams(needs_layout_passes=False),
    )(_sc_rep_sums_body)


def _sc_rep_sums_body(src_hbm, qry_hbm, emb_hbm, pos_hbm, sums_out, cnt_out,
                      tok_v, tokrd_v, rows_v, pos_v, stage_v, cstage_v, sem):
    w = _wid()

    def task(tok_flat, tok_off, n_tok, out_row):
        pltpu.sync_copy(tok_flat.at[pl.ds(tok_off, n_tok)],
                        tok_v.at[pl.ds(0, n_tok)])
        pltpu.sync_copy(tok_flat.at[pl.ds(tok_off, n_tok)],
                        tokrd_v.at[pl.ds(0, n_tok)])
        accs = tuple(jnp.zeros((LN,), jnp.float32) for _ in range(D_EMB // LN))
        cnt = jnp.zeros((LN,), jnp.float32)
        for h in range(n_tok // 128):
            pltpu.sync_copy(pos_hbm.at[pl.ds(h * 128, 128)], pos_v)
            pltpu.async_copy(emb_hbm.at[tok_v.at[pl.ds(h * 128, 128)]],
                             rows_v, sem).wait()

            def body(t, carry):
                accs_c = carry[:-1]
                cnt_c = carry[-1]
                idxs = jnp.full((LN,), h * 128 + t, dtype=jnp.int32)
                tok16 = plsc.load_gather(tokrd_v, [idxs])
                m = (tok16 > 0).astype(jnp.float32)
                new = []
                for c in range(D_EMB // LN):
                    rv = rows_v[t, pl.ds(c * LN, LN)]
                    pv = pos_v[t, pl.ds(c * LN, LN)]
                    new.append(accs_c[c] + (rv + pv) * m)
                return tuple(new) + (cnt_c + m,)

            out = lax.fori_loop(0, 128, body, accs + (cnt,))
            accs = out[:-1]
            cnt = out[-1]
        for c in range(D_EMB // LN):
            stage_v[pl.ds(c * LN, LN)] = accs[c]
        cstage_v[...] = cnt
        pltpu.sync_copy(stage_v, sums_out.at[pl.ds(out_row * D_EMB, D_EMB)])
        pltpu.sync_copy(cstage_v, cnt_out.at[pl.ds(out_row * LN, LN)])

    task(src_hbm, w * SRC, SRC, w)

    @pl.when(w < B)
    def _():
        task(qry_hbm, w * QRY, QRY, 32 + w)


@functools.cache
def _make_sc_build_gather():
    mesh = plsc.VectorSubcoreMesh(core_axis_name="c", subcore_axis_name="s")
    return functools.partial(
        pl.kernel,
        out_type=(
            jax.ShapeDtypeStruct((B * LN,), jnp.float32),
            jax.ShapeDtypeStruct((B * LN,), jnp.int32),
            jax.ShapeDtypeStruct((B * L_CAT,), jnp.int32),
            jax.ShapeDtypeStruct((B * L_CAT, D_EMB), jnp.float32),
            jax.ShapeDtypeStruct((B * TGT, D_EMB), jnp.float32),
        ),
        mesh=mesh,
        scratch_types=[
            pltpu.VMEM((5 * D_EMB,), jnp.float32),
            pltpu.VMEM((5 * LN,), jnp.float32),
            pltpu.VMEM((LN,), jnp.float32),
            pltpu.VMEM((LN,), jnp.int32),
            pltpu.VMEM((LN, SRC), jnp.int32),
            pltpu.VMEM((QRY,), jnp.int32),
            pltpu.VMEM((96,), jnp.int32),
            pltpu.VMEM((96, D_EMB), jnp.float32),
            pltpu.VMEM((LN,), jnp.int32),
            pltpu.VMEM((LN, D_EMB), jnp.float32),
            pltpu.SemaphoreType.DMA,
        ],
        compiler_params=pltpu.CompilerParams(needs_layout_passes=False),
    )(_sc_build_gather_body)


def _sc_build_gather_body(sums_hbm, cnts_hbm, src2d_hbm, qry_hbm, tgt_hbm,
                          emb_hbm, sims_out, lens_out, ns_out, x_out, y_out,
                          sums_v, cnts_v, simstage_v, lenstage_v,
                          srcsel_v, q_v, ns_v, xrows_v, tidx_v, yrows_v, sem):
    w = _wid()
    b = w // 4
    qq = w % 4

    pltpu.sync_copy(sums_hbm.at[pl.ds(b * CTX * D_EMB, CTX * D_EMB)],
                    sums_v.at[pl.ds(0, CTX * D_EMB)])
    pltpu.sync_copy(sums_hbm.at[pl.ds((32 + b) * D_EMB, D_EMB)],
                    sums_v.at[pl.ds(CTX * D_EMB, D_EMB)])
    pltpu.sync_copy(cnts_hbm.at[pl.ds(b * CTX * LN, CTX * LN)],
                    cnts_v.at[pl.ds(0, CTX * LN)])
    pltpu.sync_copy(cnts_hbm.at[pl.ds((32 + b) * LN, LN)],
                    cnts_v.at[pl.ds(CTX * LN, LN)])

    denq = jnp.maximum(cnts_v[pl.ds(CTX * LN, LN)], 1.0)
    iota = lax.iota(jnp.int32, LN)
    v = jnp.full((LN,), -1e30, dtype=jnp.float32)
    for c in range(CTX):
        denc = jnp.maximum(cnts_v[pl.ds(c * LN, LN)], 1.0)
        acc = jnp.zeros((LN,), jnp.float32)
        for k in range(D_EMB // LN):
            sv = sums_v[pl.ds(c * D_EMB + k * LN, LN)] / denc
            qv = sums_v[pl.ds(CTX * D_EMB + k * LN, LN)] / denq
            acc = acc + sv * qv
        t_c = jnp.sum(acc)
        v = jnp.where(iota == c, jnp.full((LN,), t_c), v)

    mx = jnp.max(v)
    mxv = jnp.full((LN,), mx)
    e = jnp.exp(v - mxv)
    e = jnp.where(iota < CTX, e, 0.0)
    ssum = jnp.sum(e)
    sims16 = e / jnp.full((LN,), ssum)
    pickv = plsc.all_reduce_ffs(v == mxv)

    @pl.when(qq == 0)
    def _():
        simstage_v[...] = sims16
        pltpu.sync_copy(simstage_v, sims_out.at[pl.ds(b * LN, LN)])

    pltpu.async_copy(src2d_hbm.at[b * CTX + pickv], srcsel_v, sem).wait()
    pltpu.sync_copy(qry_hbm.at[pl.ds(b * QRY, QRY)], q_v)

    sl = jnp.zeros((LN,), jnp.int32)
    for k in range(SRC // LN):
        chunk = srcsel_v[0, pl.ds(k * LN, LN)]
        sl = sl + plsc.all_reduce_population_count(chunk > 0)
    ql = jnp.zeros((LN,), jnp.int32)
    for k in range(QRY // LN):
        chunk = q_v[pl.ds(k * LN, LN)]
        ql = ql + plsc.all_reduce_population_count(chunk > 0)

    @pl.when(qq == 0)
    def _():
        lenstage_v[...] = sl + ql
        pltpu.sync_copy(lenstage_v, lens_out.at[pl.ds(b * LN, LN)])

    zeros16 = jnp.zeros((LN,), jnp.int32)
    copies = []
    for k in range(96 // LN):
        pos = iota + (qq * 96 + k * LN)
        in_src = pos < sl
        spos = jnp.minimum(pos, SRC - 1)
        s_tok = plsc.load_gather(srcsel_v, [zeros16, spos])
        qpos = jnp.clip(pos - sl, 0, QRY - 1)
        q_tok = plsc.load_gather(q_v, [qpos])
        in_q = jnp.logical_and(pos >= sl, pos < sl + ql)
        tok = jnp.where(in_src, s_tok, jnp.where(in_q, q_tok, zeros16))
        ns_v[pl.ds(k * LN, LN)] = tok
        copies.append(pltpu.async_copy(
            emb_hbm.at[tok], xrows_v.at[pl.ds(k * LN, LN)], sem))
    for c in copies:
        c.wait()

    row0 = b * L_CAT + qq * 96
    pltpu.sync_copy(ns_v, ns_out.at[pl.ds(row0, 96)])
    pltpu.sync_copy(xrows_v, x_out.at[pl.ds(row0, 96)])

    pltpu.sync_copy(tgt_hbm.at[pl.ds(w * LN, LN)], tidx_v)
    pltpu.async_copy(emb_hbm.at[tidx_v], yrows_v, sem).wait()
    pltpu.sync_copy(yrows_v, y_out.at[pl.ds(w * LN, LN)])


def _tc_scan_body(x_ref, y_ref, lens_ref, ewx_ref, ewh_ref, eb_ref,
                  dwx_ref, dwh_ref, db_ref, H_ref, Hd_ref, gx_s, gy_s,
                  htm_s, hdtm_s):
    max_len = jnp.max(lens_ref[...])
    for b in range(B):
        gx_s[:, b, :] = (jnp.dot(x_ref[b], ewx_ref[...], precision=_PREC,
                                 preferred_element_type=jnp.float32)
                         + eb_ref[...])
        gy_s[:, b, :] = (jnp.dot(y_ref[b], dwx_ref[...], precision=_PREC,
                                 preferred_element_type=jnp.float32)
                         + db_ref[...])

    def gru_step(g, h, wh_ref):
        gh = jnp.dot(h, wh_ref[...], precision=_PREC,
                     preferred_element_type=jnp.float32)
        r = jax.nn.sigmoid(g[:, :D_HID] + gh[:, :D_HID])
        z = jax.nn.sigmoid(g[:, D_HID:2 * D_HID] + gh[:, D_HID:2 * D_HID])
        n = jnp.tanh(g[:, 2 * D_HID:] + r * gh[:, 2 * D_HID:])
        return (1.0 - z) * n + z * h

    def estep(t, h):
        g = gx_s[t]
        h_new = gru_step(g, h, ewh_ref)
        h2 = jnp.where(t < max_len, h_new, h)
        htm_s[t] = h2
        return h2

    hT = lax.fori_loop(0, L_CAT, estep, jnp.zeros((B, D_HID), jnp.float32))

    def dstep(t, h):
        g = gy_s[t]
        h2 = gru_step(g, h, dwh_ref)
        hdtm_s[t] = h2
        return h2

    lax.fori_loop(0, TGT, dstep, hT)

    for b in range(B):
        H_ref[b] = htm_s[:, b, :]
        Hd_ref[b] = hdtm_s[:, b, :]


def _tc_scans(x_bm, y_bm, lens2d, ewx, ewh, eb, dwx, dwh, db, interpret=False):
    return pl.pallas_call(
        _tc_scan_body,
        out_shape=(
            jax.ShapeDtypeStruct((B, L_CAT, D_HID), jnp.float32),
            jax.ShapeDtypeStruct((B, TGT, D_HID), jnp.float32),
        ),
        scratch_shapes=[
            pltpu.VMEM((L_CAT, B, 3 * D_HID), jnp.float32),
            pltpu.VMEM((TGT, B, 3 * D_HID), jnp.float32),
            pltpu.VMEM((L_CAT, B, D_HID), jnp.float32),
            pltpu.VMEM((TGT, B, D_HID), jnp.float32),
        ],
        interpret=interpret,
    )(x_bm, y_bm, lens2d, ewx, ewh, eb, dwx, dwh, db)


def _tc_out_body(H_ref, Hd_ref, ns_ref, wa_ref, wc_ref, wo_ref, out_ref,
                 hc_s, logit_s):
    Hb = H_ref[0]
    Hd = Hd_ref[0]
    ns = ns_ref[0, 0]
    neg = jnp.where(ns > 0, 0.0, -1e9)

    hc_s[...] = jnp.tanh(jnp.dot(Hb, wc_ref[...], precision=_PREC,
                                 preferred_element_type=jnp.float32))

    tdims = (((1,), (1,)), ((), ()))
    tmp = lax.dot_general(Hd, wa_ref[...], tdims, precision=_PREC,
                          preferred_element_type=jnp.float32)
    att = lax.dot_general(tmp, Hb, tdims, precision=_PREC,
                          preferred_element_type=jnp.float32) + neg[None, :]
    am = jnp.max(att, axis=1, keepdims=True)
    ae = jnp.exp(att - am)
    alpha = ae / jnp.sum(ae, axis=1, keepdims=True)
    ctx = jnp.dot(alpha, Hb, precision=_PREC,
                  preferred_element_type=jnp.float32)

    copy_log = lax.dot_general(Hd, hc_s[...], tdims, precision=_PREC,
                               preferred_element_type=jnp.float32) + neg[None, :]
    cat = jnp.concatenate([Hd, ctx], axis=1)

    mrow = jnp.max(copy_log, axis=1, keepdims=True)
    for blk in range(NBLK):
        lo = blk * BLKV
        bw = min(BLKV, V - lo)
        gl = jnp.dot(cat, wo_ref[:, lo:lo + bw],
                     precision=_PREC, preferred_element_type=jnp.float32)
        logit_s[:, lo:lo + bw] = gl
        mrow = jnp.maximum(mrow, jnp.max(gl, axis=1, keepdims=True))

    pc = jnp.exp(copy_log - mrow)
    zrow = jnp.sum(pc, axis=1, keepdims=True)
    for blk in range(NBLK):
        lo = blk * BLKV
        bw = min(BLKV, V - lo)
        zrow = zrow + jnp.sum(
            jnp.exp(logit_s[:, lo:lo + bw] - mrow), axis=1, keepdims=True)
    inv_z = 1.0 / zrow

    for blk in range(NBLK):
        lo = blk * BLKV
        bw = min(BLKV, V - lo)
        num = jnp.exp(logit_s[:, lo:lo + bw] - mrow)
        colidx = lo + lax.broadcasted_iota(jnp.int32, (L_CAT, bw), 1)
        oh = (ns[:, None] == colidx).astype(jnp.float32)
        num = num + jnp.dot(pc, oh, precision=_PREC,
                            preferred_element_type=jnp.float32)
        out_ref[0, :, lo:lo + bw] = jnp.log(num * inv_z + 1e-10)


def _tc_out(Hb, Hdb, ns3, wa, wc, wo, interpret=False):
    return pl.pallas_call(
        _tc_out_body,
        grid=(B,),
        in_specs=[
            pl.BlockSpec((1, L_CAT, D_HID), lambda b: (b, 0, 0)),
            pl.BlockSpec((1, TGT, D_HID), lambda b: (b, 0, 0)),
            pl.BlockSpec((1, 1, L_CAT), lambda b: (b, 0, 0)),
            pl.BlockSpec((D_HID, D_HID), lambda b: (0, 0)),
            pl.BlockSpec((D_HID, D_HID), lambda b: (0, 0)),
            pl.BlockSpec((2 * D_HID, V), lambda b: (0, 0)),
        ],
        out_specs=pl.BlockSpec((1, TGT, V), lambda b: (b, 0, 0)),
        out_shape=jax.ShapeDtypeStruct((B, TGT, V), jnp.float32),
        scratch_shapes=[
            pltpu.VMEM((L_CAT, D_HID), jnp.float32),
            pltpu.VMEM((TGT, V), jnp.float32),
        ],
        compiler_params=pltpu.CompilerParams(
            dimension_semantics=("arbitrary",)),
        interpret=interpret,
    )(Hb, Hdb, ns3, wa, wc, wo)


def kernel(sources, queries, lengths, targets, emb, pos_emb,
           enc_Wx, enc_Wh, enc_b, dec_Wx, dec_Wh, dec_b, Wa, Wc, Wo):
    del lengths
    src_flat = sources.reshape(-1).astype(jnp.int32)
    qry_flat = queries.reshape(-1).astype(jnp.int32)
    tgt_flat = targets.reshape(-1).astype(jnp.int32)
    pos256 = pos_emb[:SRC]

    sums, cnts = _make_sc_rep_sums()(src_flat, qry_flat, emb, pos256)
    sims_p, lens_p, ns_flat, x_rows, y_rows = _make_sc_build_gather()(
        sums, cnts, sources.astype(jnp.int32), qry_flat, tgt_flat, emb)

    x_bm = x_rows.reshape(B, L_CAT, D_EMB)
    y_bm = y_rows.reshape(B, TGT, D_EMB)
    lens2d = lens_p.reshape(B, LN)

    Hb, Hdb = _tc_scans(
        x_bm, y_bm, lens2d, enc_Wx, enc_Wh, enc_b.reshape(1, -1),
        dec_Wx, dec_Wh, dec_b.reshape(1, -1))

    ns3 = ns_flat.reshape(B, 1, L_CAT)
    outputs = _tc_out(Hb, Hdb, ns3, Wa, Wc, Wo)
    sims = sims_p.reshape(B, LN)[:, :CTX]
    return (outputs, sims)


# speedup vs baseline: 1.0065x; 1.0065x over previous
import functools

import jax
import jax.numpy as jnp
from jax import lax
from jax.experimental import pallas as pl
from jax.experimental.pallas import tpu as pltpu
from jax.experimental.pallas import tpu_sc as plsc

V = 10000
D_EMB = 256
D_HID = 512
B = 8
CTX = 4
SRC = 256
QRY = 128
TGT = 64
L_CAT = SRC + QRY

NC = 2
NS = 16
NW = NC * NS
LN = 16

VPAD = 10240
BLKV = 1024
NBLK = VPAD // BLKV

_PREC = jax.lax.Precision.DEFAULT

def _wid():
    return lax.axis_index("s") * NC + lax.axis_index("c")


@functools.cache
def _make_sc_rep_sums():
    mesh = plsc.VectorSubcoreMesh(core_axis_name="c", subcore_axis_name="s")
    return functools.partial(
        pl.kernel,
        out_type=(
            jax.ShapeDtypeStruct((40 * D_EMB,), jnp.float32),
            jax.ShapeDtypeStruct((40 * LN,), jnp.float32),
        ),
        mesh=mesh,
        scratch_types=[
            pltpu.VMEM((SRC,), jnp.int32),
            pltpu.VMEM((SRC,), jnp.int32),
            pltpu.VMEM((128, D_EMB), jnp.float32),
            pltpu.VMEM((128, D_EMB), jnp.float32),
            pltpu.VMEM((D_EMB,), jnp.float32),
            pltpu.VMEM((LN,), jnp.float32),
            pltpu.SemaphoreType.DMA,
        ],
        compiler_params=pltpu.CompilerParams(needs_layout_passes=False),
    )(_sc_rep_sums_body)


def _sc_rep_sums_body(src_hbm, qry_hbm, emb_hbm, pos_hbm, sums_out, cnt_out,
                      tok_v, tokrd_v, rows_v, pos_v, stage_v, cstage_v, sem):
    w = _wid()

    def task(tok_flat, tok_off, n_tok, out_row):
        pltpu.sync_copy(tok_flat.at[pl.ds(tok_off, n_tok)],
                        tok_v.at[pl.ds(0, n_tok)])
        pltpu.sync_copy(tok_flat.at[pl.ds(tok_off, n_tok)],
                        tokrd_v.at[pl.ds(0, n_tok)])
        accs = tuple(jnp.zeros((LN,), jnp.float32) for _ in range(D_EMB // LN))
        cnt = jnp.zeros((LN,), jnp.float32)
        for h in range(n_tok // 128):
            pltpu.sync_copy(pos_hbm.at[pl.ds(h * 128, 128)], pos_v)
            pltpu.async_copy(emb_hbm.at[tok_v.at[pl.ds(h * 128, 128)]],
                             rows_v, sem).wait()

            def body(t, carry):
                accs_c = carry[:-1]
                cnt_c = carry[-1]
                idxs = jnp.full((LN,), h * 128 + t, dtype=jnp.int32)
                tok16 = plsc.load_gather(tokrd_v, [idxs])
                m = (tok16 > 0).astype(jnp.float32)
                new = []
                for c in range(D_EMB // LN):
                    rv = rows_v[t, pl.ds(c * LN, LN)]
                    pv = pos_v[t, pl.ds(c * LN, LN)]
                    new.append(accs_c[c] + (rv + pv) * m)
                return tuple(new) + (cnt_c + m,)

            out = lax.fori_loop(0, 128, body, accs + (cnt,))
            accs = out[:-1]
            cnt = out[-1]
        for c in range(D_EMB // LN):
            stage_v[pl.ds(c * LN, LN)] = accs[c]
        cstage_v[...] = cnt
        pltpu.sync_copy(stage_v, sums_out.at[pl.ds(out_row * D_EMB, D_EMB)])
        pltpu.sync_copy(cstage_v, cnt_out.at[pl.ds(out_row * LN, LN)])

    task(src_hbm, w * SRC, SRC, w)

    @pl.when(w < B)
    def _():
        task(qry_hbm, w * QRY, QRY, 32 + w)


@functools.cache
def _make_sc_build_gather():
    mesh = plsc.VectorSubcoreMesh(core_axis_name="c", subcore_axis_name="s")
    return functools.partial(
        pl.kernel,
        out_type=(
            jax.ShapeDtypeStruct((B * LN,), jnp.float32),
            jax.ShapeDtypeStruct((B * LN,), jnp.int32),
            jax.ShapeDtypeStruct((B * L_CAT,), jnp.int32),
            jax.ShapeDtypeStruct((B * L_CAT, D_EMB), jnp.float32),
            jax.ShapeDtypeStruct((B * TGT, D_EMB), jnp.float32),
        ),
        mesh=mesh,
        scratch_types=[
            pltpu.VMEM((5 * D_EMB,), jnp.float32),
            pltpu.VMEM((5 * LN,), jnp.float32),
            pltpu.VMEM((LN,), jnp.float32),
            pltpu.VMEM((LN,), jnp.int32),
            pltpu.VMEM((LN, SRC), jnp.int32),
            pltpu.VMEM((QRY,), jnp.int32),
            pltpu.VMEM((96,), jnp.int32),
            pltpu.VMEM((96, D_EMB), jnp.float32),
            pltpu.VMEM((LN,), jnp.int32),
            pltpu.VMEM((LN, D_EMB), jnp.float32),
            pltpu.SemaphoreType.DMA,
        ],
        compiler_params=pltpu.CompilerParams(needs_layout_passes=False),
    )(_sc_build_gather_body)


def _sc_build_gather_body(sums_hbm, cnts_hbm, src2d_hbm, qry_hbm, tgt_hbm,
                          emb_hbm, sims_out, lens_out, ns_out, x_out, y_out,
                          sums_v, cnts_v, simstage_v, lenstage_v,
                          srcsel_v, q_v, ns_v, xrows_v, tidx_v, yrows_v, sem):
    w = _wid()
    b = w // 4
    qq = w % 4

    pltpu.sync_copy(sums_hbm.at[pl.ds(b * CTX * D_EMB, CTX * D_EMB)],
                    sums_v.at[pl.ds(0, CTX * D_EMB)])
    pltpu.sync_copy(sums_hbm.at[pl.ds((32 + b) * D_EMB, D_EMB)],
                    sums_v.at[pl.ds(CTX * D_EMB, D_EMB)])
    pltpu.sync_copy(cnts_hbm.at[pl.ds(b * CTX * LN, CTX * LN)],
                    cnts_v.at[pl.ds(0, CTX * LN)])
    pltpu.sync_copy(cnts_hbm.at[pl.ds((32 + b) * LN, LN)],
                    cnts_v.at[pl.ds(CTX * LN, LN)])

    denq = jnp.maximum(cnts_v[pl.ds(CTX * LN, LN)], 1.0)
    iota = lax.iota(jnp.int32, LN)
    v = jnp.full((LN,), -1e30, dtype=jnp.float32)
    for c in range(CTX):
        denc = jnp.maximum(cnts_v[pl.ds(c * LN, LN)], 1.0)
        acc = jnp.zeros((LN,), jnp.float32)
        for k in range(D_EMB // LN):
            sv = sums_v[pl.ds(c * D_EMB + k * LN, LN)] / denc
            qv = sums_v[pl.ds(CTX * D_EMB + k * LN, LN)] / denq
            acc = acc + sv * qv
        t_c = jnp.sum(acc)
        v = jnp.where(iota == c, jnp.full((LN,), t_c), v)

    mx = jnp.max(v)
    mxv = jnp.full((LN,), mx)
    e = jnp.exp(v - mxv)
    e = jnp.where(iota < CTX, e, 0.0)
    ssum = jnp.sum(e)
    sims16 = e / jnp.full((LN,), ssum)
    pickv = plsc.all_reduce_ffs(v == mxv)

    @pl.when(qq == 0)
    def _():
        simstage_v[...] = sims16
        pltpu.sync_copy(simstage_v, sims_out.at[pl.ds(b * LN, LN)])

    pltpu.async_copy(src2d_hbm.at[b * CTX + pickv], srcsel_v, sem).wait()
    pltpu.sync_copy(qry_hbm.at[pl.ds(b * QRY, QRY)], q_v)

    sl = jnp.zeros((LN,), jnp.int32)
    for k in range(SRC // LN):
        chunk = srcsel_v[0, pl.ds(k * LN, LN)]
        sl = sl + plsc.all_reduce_population_count(chunk > 0)
    ql = jnp.zeros((LN,), jnp.int32)
    for k in range(QRY // LN):
        chunk = q_v[pl.ds(k * LN, LN)]
        ql = ql + plsc.all_reduce_population_count(chunk > 0)

    @pl.when(qq == 0)
    def _():
        lenstage_v[...] = sl + ql
        pltpu.sync_copy(lenstage_v, lens_out.at[pl.ds(b * LN, LN)])

    zeros16 = jnp.zeros((LN,), jnp.int32)
    copies = []
    for k in range(96 // LN):
        pos = iota + (qq * 96 + k * LN)
        in_src = pos < sl
        spos = jnp.minimum(pos, SRC - 1)
        s_tok = plsc.load_gather(srcsel_v, [zeros16, spos])
        qpos = jnp.clip(pos - sl, 0, QRY - 1)
        q_tok = plsc.load_gather(q_v, [qpos])
        in_q = jnp.logical_and(pos >= sl, pos < sl + ql)
        tok = jnp.where(in_src, s_tok, jnp.where(in_q, q_tok, zeros16))
        ns_v[pl.ds(k * LN, LN)] = tok
        copies.append(pltpu.async_copy(
            emb_hbm.at[tok], xrows_v.at[pl.ds(k * LN, LN)], sem))
    for c in copies:
        c.wait()

    row0 = b * L_CAT + qq * 96
    pltpu.sync_copy(ns_v, ns_out.at[pl.ds(row0, 96)])
    pltpu.sync_copy(xrows_v, x_out.at[pl.ds(row0, 96)])

    pltpu.sync_copy(tgt_hbm.at[pl.ds(w * LN, LN)], tidx_v)
    pltpu.async_copy(emb_hbm.at[tidx_v], yrows_v, sem).wait()
    pltpu.sync_copy(yrows_v, y_out.at[pl.ds(w * LN, LN)])


def _tc_scan_body(x_ref, y_ref, lens_ref, ewx_ref, ewh_ref, eb_ref,
                  dwx_ref, dwh_ref, db_ref, H_ref, Hd_ref, gx_s, gy_s):
    max_len = jnp.max(lens_ref[...])
    for b in range(B):
        gx_s[b] = (jnp.dot(x_ref[b], ewx_ref[...], precision=_PREC,
                           preferred_element_type=jnp.float32) + eb_ref[...])
        gy_s[b] = (jnp.dot(y_ref[b], dwx_ref[...], precision=_PREC,
                           preferred_element_type=jnp.float32) + db_ref[...])

    ewh_bf = ewh_ref[...].astype(jnp.bfloat16)
    dwh_bf = dwh_ref[...].astype(jnp.bfloat16)

    def gru_step(g, h, wh_bf):
        gh = jnp.dot(h.astype(jnp.bfloat16), wh_bf,
                     preferred_element_type=jnp.float32)
        r = jax.nn.sigmoid(g[:, :D_HID] + gh[:, :D_HID])
        z = jax.nn.sigmoid(g[:, D_HID:2 * D_HID] + gh[:, D_HID:2 * D_HID])
        n = jnp.tanh(g[:, 2 * D_HID:] + r * gh[:, 2 * D_HID:])
        return (1.0 - z) * n + z * h

    def estep(t, h):
        g = gx_s[:, t, :]
        h_new = gru_step(g, h, ewh_bf)
        h2 = jnp.where(t < max_len, h_new, h)
        H_ref[:, t, :] = h2
        return h2

    hT = lax.fori_loop(0, L_CAT, estep, jnp.zeros((B, D_HID), jnp.float32))

    def dstep(t, h):
        g = gy_s[:, t, :]
        h2 = gru_step(g, h, dwh_bf)
        Hd_ref[:, t, :] = h2
        return h2

    lax.fori_loop(0, TGT, dstep, hT)


def _tc_scans(x_bm, y_bm, lens2d, ewx, ewh, eb, dwx, dwh, db, interpret=False):
    return pl.pallas_call(
        _tc_scan_body,
        out_shape=(
            jax.ShapeDtypeStruct((B, L_CAT, D_HID), jnp.float32),
            jax.ShapeDtypeStruct((B, TGT, D_HID), jnp.float32),
        ),
        scratch_shapes=[
            pltpu.VMEM((B, L_CAT, 3 * D_HID), jnp.float32),
            pltpu.VMEM((B, TGT, 3 * D_HID), jnp.float32),
        ],
        interpret=interpret,
    )(x_bm, y_bm, lens2d, ewx, ewh, eb, dwx, dwh, db)


def _tc_out_body(H_ref, Hd_ref, ns_ref, wa_ref, wc_ref, wo_ref, out_ref,
                 hc_s, logit_s):
    Hb = H_ref[0]
    Hd = Hd_ref[0]
    ns = ns_ref[0, 0]
    neg = jnp.where(ns > 0, 0.0, -1e9)

    hc_s[...] = jnp.tanh(jnp.dot(Hb, wc_ref[...], precision=_PREC,
                                 preferred_element_type=jnp.float32))

    tdims = (((1,), (1,)), ((), ()))
    tmp = lax.dot_general(Hd, wa_ref[...], tdims, precision=_PREC,
                          preferred_element_type=jnp.float32)
    att = lax.dot_general(tmp, Hb, tdims, precision=_PREC,
                          preferred_element_type=jnp.float32) + neg[None, :]
    am = jnp.max(att, axis=1, keepdims=True)
    ae = jnp.exp(att - am)
    alpha = ae / jnp.sum(ae, axis=1, keepdims=True)
    ctx = jnp.dot(alpha, Hb, precision=_PREC,
                  preferred_element_type=jnp.float32)

    copy_log = lax.dot_general(Hd, hc_s[...], tdims, precision=_PREC,
                               preferred_element_type=jnp.float32) + neg[None, :]
    cat = jnp.concatenate([Hd, ctx], axis=1)

    mrow = jnp.max(copy_log, axis=1, keepdims=True)
    for blk in range(NBLK):
        lo = blk * BLKV
        bw = min(BLKV, V - lo)
        gl = jnp.dot(cat, wo_ref[:, lo:lo + bw],
                     precision=_PREC, preferred_element_type=jnp.float32)
        logit_s[:, lo:lo + bw] = gl
        mrow = jnp.maximum(mrow, jnp.max(gl, axis=1, keepdims=True))

    pc = jnp.exp(copy_log - mrow)
    zrow = jnp.sum(pc, axis=1, keepdims=True)
    for blk in range(NBLK):
        lo = blk * BLKV
        bw = min(BLKV, V - lo)
        zrow = zrow + jnp.sum(
            jnp.exp(logit_s[:, lo:lo + bw] - mrow), axis=1, keepdims=True)
    inv_z = 1.0 / zrow

    for blk in range(NBLK):
        lo = blk * BLKV
        bw = min(BLKV, V - lo)
        num = jnp.exp(logit_s[:, lo:lo + bw] - mrow)
        colidx = lo + lax.broadcasted_iota(jnp.int32, (L_CAT, bw), 1)
        oh = (ns[:, None] == colidx).astype(jnp.float32)
        num = num + jnp.dot(pc, oh, precision=_PREC,
                            preferred_element_type=jnp.float32)
        out_ref[0, :, lo:lo + bw] = jnp.log(num * inv_z + 1e-10)


def _tc_out(Hb, Hdb, ns3, wa, wc, wo, interpret=False):
    return pl.pallas_call(
        _tc_out_body,
        grid=(B,),
        in_specs=[
            pl.BlockSpec((1, L_CAT, D_HID), lambda b: (b, 0, 0)),
            pl.BlockSpec((1, TGT, D_HID), lambda b: (b, 0, 0)),
            pl.BlockSpec((1, 1, L_CAT), lambda b: (b, 0, 0)),
            pl.BlockSpec((D_HID, D_HID), lambda b: (0, 0)),
            pl.BlockSpec((D_HID, D_HID), lambda b: (0, 0)),
            pl.BlockSpec((2 * D_HID, V), lambda b: (0, 0)),
        ],
        out_specs=pl.BlockSpec((1, TGT, V), lambda b: (b, 0, 0)),
        out_shape=jax.ShapeDtypeStruct((B, TGT, V), jnp.float32),
        scratch_shapes=[
            pltpu.VMEM((L_CAT, D_HID), jnp.float32),
            pltpu.VMEM((TGT, V), jnp.float32),
        ],
        compiler_params=pltpu.CompilerParams(
            dimension_semantics=("arbitrary",)),
        interpret=interpret,
    )(Hb, Hdb, ns3, wa, wc, wo)


def kernel(sources, queries, lengths, targets, emb, pos_emb,
           enc_Wx, enc_Wh, enc_b, dec_Wx, dec_Wh, dec_b, Wa, Wc, Wo):
    del lengths
    src_flat = sources.reshape(-1).astype(jnp.int32)
    qry_flat = queries.reshape(-1).astype(jnp.int32)
    tgt_flat = targets.reshape(-1).astype(jnp.int32)
    pos256 = pos_emb[:SRC]

    sums, cnts = _make_sc_rep_sums()(src_flat, qry_flat, emb, pos256)
    sims_p, lens_p, ns_flat, x_rows, y_rows = _make_sc_build_gather()(
        sums, cnts, sources.astype(jnp.int32), qry_flat, tgt_flat, emb)

    x_bm = x_rows.reshape(B, L_CAT, D_EMB)
    y_bm = y_rows.reshape(B, TGT, D_EMB)
    lens2d = lens_p.reshape(B, LN)

    Hb, Hdb = _tc_scans(
        x_bm, y_bm, lens2d, enc_Wx, enc_Wh, enc_b.reshape(1, -1),
        dec_Wx, dec_Wh, dec_b.reshape(1, -1))

    ns3 = ns_flat.reshape(B, 1, L_CAT)
    outputs = _tc_out(Hb, Hdb, ns3, Wa, Wc, Wo)
    sims = sims_p.reshape(B, LN)[:, :CTX]
    return (outputs, sims)


# speedup vs baseline: 1.0565x; 1.0497x over previous
import functools

import jax
import jax.numpy as jnp
from jax import lax
from jax.experimental import pallas as pl
from jax.experimental.pallas import tpu as pltpu
from jax.experimental.pallas import tpu_sc as plsc

V = 10000
D_EMB = 256
D_HID = 512
B = 8
CTX = 4
SRC = 256
QRY = 128
TGT = 64
L_CAT = SRC + QRY

NC = 2
NS = 16
NW = NC * NS
LN = 16

VPAD = 10240
BLKV = 1024
NBLK = VPAD // BLKV

_PREC = jax.lax.Precision.DEFAULT

def _wid():
    return lax.axis_index("s") * NC + lax.axis_index("c")


@functools.cache
def _make_sc_rep_sums():
    mesh = plsc.VectorSubcoreMesh(core_axis_name="c", subcore_axis_name="s")
    return functools.partial(
        pl.kernel,
        out_type=(
            jax.ShapeDtypeStruct((40 * D_EMB,), jnp.float32),
            jax.ShapeDtypeStruct((40 * LN,), jnp.float32),
        ),
        mesh=mesh,
        scratch_types=[
            pltpu.VMEM((SRC,), jnp.int32),
            pltpu.VMEM((SRC,), jnp.int32),
            pltpu.VMEM((128, D_EMB), jnp.float32),
            pltpu.VMEM((128, D_EMB), jnp.float32),
            pltpu.VMEM((D_EMB,), jnp.float32),
            pltpu.VMEM((LN,), jnp.float32),
            pltpu.SemaphoreType.DMA,
        ],
        compiler_params=pltpu.CompilerParams(needs_layout_passes=False),
    )(_sc_rep_sums_body)


def _sc_rep_sums_body(src_hbm, qry_hbm, emb_hbm, pos_hbm, sums_out, cnt_out,
                      tok_v, tokrd_v, rows_v, pos_v, stage_v, cstage_v, sem):
    w = _wid()

    def task(tok_flat, tok_off, n_tok, out_row):
        pltpu.sync_copy(tok_flat.at[pl.ds(tok_off, n_tok)],
                        tok_v.at[pl.ds(0, n_tok)])
        pltpu.sync_copy(tok_flat.at[pl.ds(tok_off, n_tok)],
                        tokrd_v.at[pl.ds(0, n_tok)])
        accs = tuple(jnp.zeros((LN,), jnp.float32) for _ in range(D_EMB // LN))
        cnt = jnp.zeros((LN,), jnp.float32)
        for h in range(n_tok // 128):
            pltpu.sync_copy(pos_hbm.at[pl.ds(h * 128, 128)], pos_v)
            pltpu.async_copy(emb_hbm.at[tok_v.at[pl.ds(h * 128, 128)]],
                             rows_v, sem).wait()

            def body(t, carry):
                accs_c = carry[:-1]
                cnt_c = carry[-1]
                idxs = jnp.full((LN,), h * 128 + t, dtype=jnp.int32)
                tok16 = plsc.load_gather(tokrd_v, [idxs])
                m = (tok16 > 0).astype(jnp.float32)
                new = []
                for c in range(D_EMB // LN):
                    rv = rows_v[t, pl.ds(c * LN, LN)]
                    pv = pos_v[t, pl.ds(c * LN, LN)]
                    new.append(accs_c[c] + (rv + pv) * m)
                return tuple(new) + (cnt_c + m,)

            out = lax.fori_loop(0, 128, body, accs + (cnt,))
            accs = out[:-1]
            cnt = out[-1]
        for c in range(D_EMB // LN):
            stage_v[pl.ds(c * LN, LN)] = accs[c]
        cstage_v[...] = cnt
        pltpu.sync_copy(stage_v, sums_out.at[pl.ds(out_row * D_EMB, D_EMB)])
        pltpu.sync_copy(cstage_v, cnt_out.at[pl.ds(out_row * LN, LN)])

    task(src_hbm, w * SRC, SRC, w)

    @pl.when(w < B)
    def _():
        task(qry_hbm, w * QRY, QRY, 32 + w)


@functools.cache
def _make_sc_build_gather():
    mesh = plsc.VectorSubcoreMesh(core_axis_name="c", subcore_axis_name="s")
    return functools.partial(
        pl.kernel,
        out_type=(
            jax.ShapeDtypeStruct((B * LN,), jnp.float32),
            jax.ShapeDtypeStruct((B * LN,), jnp.int32),
            jax.ShapeDtypeStruct((B * L_CAT,), jnp.int32),
            jax.ShapeDtypeStruct((B * L_CAT, D_EMB), jnp.float32),
            jax.ShapeDtypeStruct((B * TGT, D_EMB), jnp.float32),
        ),
        mesh=mesh,
        scratch_types=[
            pltpu.VMEM((5 * D_EMB,), jnp.float32),
            pltpu.VMEM((5 * LN,), jnp.float32),
            pltpu.VMEM((LN,), jnp.float32),
            pltpu.VMEM((LN,), jnp.int32),
            pltpu.VMEM((LN, SRC), jnp.int32),
            pltpu.VMEM((QRY,), jnp.int32),
            pltpu.VMEM((96,), jnp.int32),
            pltpu.VMEM((96, D_EMB), jnp.float32),
            pltpu.VMEM((LN,), jnp.int32),
            pltpu.VMEM((LN, D_EMB), jnp.float32),
            pltpu.SemaphoreType.DMA,
        ],
        compiler_params=pltpu.CompilerParams(needs_layout_passes=False),
    )(_sc_build_gather_body)


def _sc_build_gather_body(sums_hbm, cnts_hbm, src2d_hbm, qry_hbm, tgt_hbm,
                          emb_hbm, sims_out, lens_out, ns_out, x_out, y_out,
                          sums_v, cnts_v, simstage_v, lenstage_v,
                          srcsel_v, q_v, ns_v, xrows_v, tidx_v, yrows_v, sem):
    w = _wid()
    b = w // 4
    qq = w % 4

    pltpu.sync_copy(sums_hbm.at[pl.ds(b * CTX * D_EMB, CTX * D_EMB)],
                    sums_v.at[pl.ds(0, CTX * D_EMB)])
    pltpu.sync_copy(sums_hbm.at[pl.ds((32 + b) * D_EMB, D_EMB)],
                    sums_v.at[pl.ds(CTX * D_EMB, D_EMB)])
    pltpu.sync_copy(cnts_hbm.at[pl.ds(b * CTX * LN, CTX * LN)],
                    cnts_v.at[pl.ds(0, CTX * LN)])
    pltpu.sync_copy(cnts_hbm.at[pl.ds((32 + b) * LN, LN)],
                    cnts_v.at[pl.ds(CTX * LN, LN)])

    denq = jnp.maximum(cnts_v[pl.ds(CTX * LN, LN)], 1.0)
    iota = lax.iota(jnp.int32, LN)
    v = jnp.full((LN,), -1e30, dtype=jnp.float32)
    for c in range(CTX):
        denc = jnp.maximum(cnts_v[pl.ds(c * LN, LN)], 1.0)
        acc = jnp.zeros((LN,), jnp.float32)
        for k in range(D_EMB // LN):
            sv = sums_v[pl.ds(c * D_EMB + k * LN, LN)] / denc
            qv = sums_v[pl.ds(CTX * D_EMB + k * LN, LN)] / denq
            acc = acc + sv * qv
        t_c = jnp.sum(acc)
        v = jnp.where(iota == c, jnp.full((LN,), t_c), v)

    mx = jnp.max(v)
    mxv = jnp.full((LN,), mx)
    e = jnp.exp(v - mxv)
    e = jnp.where(iota < CTX, e, 0.0)
    ssum = jnp.sum(e)
    sims16 = e / jnp.full((LN,), ssum)
    pickv = plsc.all_reduce_ffs(v == mxv)

    @pl.when(qq == 0)
    def _():
        simstage_v[...] = sims16
        pltpu.sync_copy(simstage_v, sims_out.at[pl.ds(b * LN, LN)])

    pltpu.async_copy(src2d_hbm.at[b * CTX + pickv], srcsel_v, sem).wait()
    pltpu.sync_copy(qry_hbm.at[pl.ds(b * QRY, QRY)], q_v)

    sl = jnp.zeros((LN,), jnp.int32)
    for k in range(SRC // LN):
        chunk = srcsel_v[0, pl.ds(k * LN, LN)]
        sl = sl + plsc.all_reduce_population_count(chunk > 0)
    ql = jnp.zeros((LN,), jnp.int32)
    for k in range(QRY // LN):
        chunk = q_v[pl.ds(k * LN, LN)]
        ql = ql + plsc.all_reduce_population_count(chunk > 0)

    @pl.when(qq == 0)
    def _():
        lenstage_v[...] = sl + ql
        pltpu.sync_copy(lenstage_v, lens_out.at[pl.ds(b * LN, LN)])

    zeros16 = jnp.zeros((LN,), jnp.int32)
    copies = []
    for k in range(96 // LN):
        pos = iota + (qq * 96 + k * LN)
        in_src = pos < sl
        spos = jnp.minimum(pos, SRC - 1)
        s_tok = plsc.load_gather(srcsel_v, [zeros16, spos])
        qpos = jnp.clip(pos - sl, 0, QRY - 1)
        q_tok = plsc.load_gather(q_v, [qpos])
        in_q = jnp.logical_and(pos >= sl, pos < sl + ql)
        tok = jnp.where(in_src, s_tok, jnp.where(in_q, q_tok, zeros16))
        ns_v[pl.ds(k * LN, LN)] = tok
        copies.append(pltpu.async_copy(
            emb_hbm.at[tok], xrows_v.at[pl.ds(k * LN, LN)], sem))
    for c in copies:
        c.wait()

    row0 = b * L_CAT + qq * 96
    pltpu.sync_copy(ns_v, ns_out.at[pl.ds(row0, 96)])
    pltpu.sync_copy(xrows_v, x_out.at[pl.ds(row0, 96)])

    pltpu.sync_copy(tgt_hbm.at[pl.ds(w * LN, LN)], tidx_v)
    pltpu.async_copy(emb_hbm.at[tidx_v], yrows_v, sem).wait()
    pltpu.sync_copy(yrows_v, y_out.at[pl.ds(w * LN, LN)])


def _tc_scan_body(x_ref, y_ref, lens_ref, ewx_ref, ewh_ref, eb_ref,
                  dwx_ref, dwh_ref, db_ref, H_ref, Hd_ref, gx_s, gy_s):
    max_len = jnp.max(lens_ref[...])
    for b in range(B):
        gx_s[b] = (jnp.dot(x_ref[b], ewx_ref[...], precision=_PREC,
                           preferred_element_type=jnp.float32) + eb_ref[...])
        gy_s[b] = (jnp.dot(y_ref[b], dwx_ref[...], precision=_PREC,
                           preferred_element_type=jnp.float32) + db_ref[...])

    def gru_step(g, h, wh_ref):
        gh = jnp.dot(h, wh_ref[...], precision=_PREC,
                     preferred_element_type=jnp.float32)
        r = jax.nn.sigmoid(g[:, :D_HID] + gh[:, :D_HID])
        z = jax.nn.sigmoid(g[:, D_HID:2 * D_HID] + gh[:, D_HID:2 * D_HID])
        n = jnp.tanh(g[:, 2 * D_HID:] + r * gh[:, 2 * D_HID:])
        return (1.0 - z) * n + z * h

    def estep(t, h):
        g = gx_s[:, t, :]
        h_new = gru_step(g, h, ewh_ref)
        h2 = jnp.where(t < max_len, h_new, h)
        H_ref[:, t, :] = h2
        return h2

    hT = lax.fori_loop(0, L_CAT, estep, jnp.zeros((B, D_HID), jnp.float32),
                       unroll=4)

    def dstep(t, h):
        g = gy_s[:, t, :]
        h2 = gru_step(g, h, dwh_ref)
        Hd_ref[:, t, :] = h2
        return h2

    lax.fori_loop(0, TGT, dstep, hT, unroll=4)


def _tc_scans(x_bm, y_bm, lens2d, ewx, ewh, eb, dwx, dwh, db, interpret=False):
    return pl.pallas_call(
        _tc_scan_body,
        out_shape=(
            jax.ShapeDtypeStruct((B, L_CAT, D_HID), jnp.float32),
            jax.ShapeDtypeStruct((B, TGT, D_HID), jnp.float32),
        ),
        scratch_shapes=[
            pltpu.VMEM((B, L_CAT, 3 * D_HID), jnp.float32),
            pltpu.VMEM((B, TGT, 3 * D_HID), jnp.float32),
        ],
        interpret=interpret,
    )(x_bm, y_bm, lens2d, ewx, ewh, eb, dwx, dwh, db)


def _tc_out_body(H_ref, Hd_ref, ns_ref, wa_ref, wc_ref, wo_ref, out_ref,
                 hc_s, logit_s):
    Hb = H_ref[0]
    Hd = Hd_ref[0]
    ns = ns_ref[0, 0]
    neg = jnp.where(ns > 0, 0.0, -1e9)

    hc_s[...] = jnp.tanh(jnp.dot(Hb, wc_ref[...], precision=_PREC,
                                 preferred_element_type=jnp.float32))

    tdims = (((1,), (1,)), ((), ()))
    tmp = lax.dot_general(Hd, wa_ref[...], tdims, precision=_PREC,
                          preferred_element_type=jnp.float32)
    att = lax.dot_general(tmp, Hb, tdims, precision=_PREC,
                          preferred_element_type=jnp.float32) + neg[None, :]
    am = jnp.max(att, axis=1, keepdims=True)
    ae = jnp.exp(att - am)
    alpha = ae / jnp.sum(ae, axis=1, keepdims=True)
    ctx = jnp.dot(alpha, Hb, precision=_PREC,
                  preferred_element_type=jnp.float32)

    copy_log = lax.dot_general(Hd, hc_s[...], tdims, precision=_PREC,
                               preferred_element_type=jnp.float32) + neg[None, :]
    cat = jnp.concatenate([Hd, ctx], axis=1)

    mrow = jnp.max(copy_log, axis=1, keepdims=True)
    for blk in range(NBLK):
        lo = blk * BLKV
        bw = min(BLKV, V - lo)
        gl = jnp.dot(cat, wo_ref[:, lo:lo + bw],
                     precision=_PREC, preferred_element_type=jnp.float32)
        logit_s[:, lo:lo + bw] = gl
        mrow = jnp.maximum(mrow, jnp.max(gl, axis=1, keepdims=True))

    pc = jnp.exp(copy_log - mrow)
    zrow = jnp.sum(pc, axis=1, keepdims=True)
    for blk in range(NBLK):
        lo = blk * BLKV
        bw = min(BLKV, V - lo)
        zrow = zrow + jnp.sum(
            jnp.exp(logit_s[:, lo:lo + bw] - mrow), axis=1, keepdims=True)
    inv_z = 1.0 / zrow

    for blk in range(NBLK):
        lo = blk * BLKV
        bw = min(BLKV, V - lo)
        num = jnp.exp(logit_s[:, lo:lo + bw] - mrow)
        colidx = lo + lax.broadcasted_iota(jnp.int32, (L_CAT, bw), 1)
        oh = (ns[:, None] == colidx).astype(jnp.float32)
        num = num + jnp.dot(pc, oh, precision=_PREC,
                            preferred_element_type=jnp.float32)
        out_ref[0, :, lo:lo + bw] = jnp.log(num * inv_z + 1e-10)


def _tc_out(Hb, Hdb, ns3, wa, wc, wo, interpret=False):
    return pl.pallas_call(
        _tc_out_body,
        grid=(B,),
        in_specs=[
            pl.BlockSpec((1, L_CAT, D_HID), lambda b: (b, 0, 0)),
            pl.BlockSpec((1, TGT, D_HID), lambda b: (b, 0, 0)),
            pl.BlockSpec((1, 1, L_CAT), lambda b: (b, 0, 0)),
            pl.BlockSpec((D_HID, D_HID), lambda b: (0, 0)),
            pl.BlockSpec((D_HID, D_HID), lambda b: (0, 0)),
            pl.BlockSpec((2 * D_HID, V), lambda b: (0, 0)),
        ],
        out_specs=pl.BlockSpec((1, TGT, V), lambda b: (b, 0, 0)),
        out_shape=jax.ShapeDtypeStruct((B, TGT, V), jnp.float32),
        scratch_shapes=[
            pltpu.VMEM((L_CAT, D_HID), jnp.float32),
            pltpu.VMEM((TGT, V), jnp.float32),
        ],
        compiler_params=pltpu.CompilerParams(
            dimension_semantics=("arbitrary",)),
        interpret=interpret,
    )(Hb, Hdb, ns3, wa, wc, wo)


def kernel(sources, queries, lengths, targets, emb, pos_emb,
           enc_Wx, enc_Wh, enc_b, dec_Wx, dec_Wh, dec_b, Wa, Wc, Wo):
    del lengths
    src_flat = sources.reshape(-1).astype(jnp.int32)
    qry_flat = queries.reshape(-1).astype(jnp.int32)
    tgt_flat = targets.reshape(-1).astype(jnp.int32)
    pos256 = pos_emb[:SRC]

    sums, cnts = _make_sc_rep_sums()(src_flat, qry_flat, emb, pos256)
    sims_p, lens_p, ns_flat, x_rows, y_rows = _make_sc_build_gather()(
        sums, cnts, sources.astype(jnp.int32), qry_flat, tgt_flat, emb)

    x_bm = x_rows.reshape(B, L_CAT, D_EMB)
    y_bm = y_rows.reshape(B, TGT, D_EMB)
    lens2d = lens_p.reshape(B, LN)

    Hb, Hdb = _tc_scans(
        x_bm, y_bm, lens2d, enc_Wx, enc_Wh, enc_b.reshape(1, -1),
        dec_Wx, dec_Wh, dec_b.reshape(1, -1))

    ns3 = ns_flat.reshape(B, 1, L_CAT)
    outputs = _tc_out(Hb, Hdb, ns3, Wa, Wc, Wo)
    sims = sims_p.reshape(B, LN)[:, :CTX]
    return (outputs, sims)
